# baseline (speedup 1.0000x reference)
import jax
import jax.numpy as jnp
from jax.experimental import pallas as pl
from jax.experimental.pallas import tpu as pltpu

_E = 16
_C = 320
_D = 1024
_N = 4096
_F = _E * _C
_BLK = 256


def _router_body(x_ref, w_ref, disp_ref, comb_ref, counts_ref):
    blk = x_ref.shape[0]

    @pl.when(pl.program_id(0) == 0)
    def _init():
        counts_ref[...] = jnp.zeros_like(counts_ref)

    logits = jnp.dot(x_ref[...], w_ref[...], preferred_element_type=jnp.float32)
    m = jnp.max(logits, axis=-1, keepdims=True)
    e = jnp.exp(logits - m)
    probs = e / jnp.sum(e, axis=-1, keepdims=True)
    gate = jnp.max(probs, axis=-1, keepdims=True)
    iota_e = jax.lax.broadcasted_iota(jnp.int32, (blk, _E), 1)
    expert = jnp.min(jnp.where(probs == gate, iota_e, _E), axis=-1, keepdims=True)
    mask = (iota_e == expert).astype(jnp.float32)

    r = jax.lax.broadcasted_iota(jnp.int32, (blk, blk), 0)
    c = jax.lax.broadcasted_iota(jnp.int32, (blk, blk), 1)
    tril = (r >= c).astype(jnp.float32)
    csum = jnp.dot(tril, mask, preferred_element_type=jnp.float32)
    pos_all = csum + counts_ref[...]
    counts_ref[...] = counts_ref[...] + csum[blk - 1 : blk, :]
    pos = jnp.sum(pos_all * mask, axis=-1, keepdims=True)
    keep = pos < float(_C)
    flat = jnp.where(keep, expert * _C + pos.astype(jnp.int32), -1)

    iota_f = jax.lax.broadcasted_iota(jnp.int32, (blk, _F), 1)
    eq = (iota_f == flat).astype(jnp.float32)
    disp_ref[...] = eq
    comb_ref[...] = eq * gate


def kernel(inputs, W):
    disp, comb = pl.pallas_call(
        _router_body,
        grid=(_N // _BLK,),
        in_specs=[
            pl.BlockSpec((_BLK, _D), lambda i: (i, 0)),
            pl.BlockSpec((_D, _E), lambda i: (0, 0)),
        ],
        out_specs=[
            pl.BlockSpec((_BLK, _F), lambda i: (i, 0)),
            pl.BlockSpec((_BLK, _F), lambda i: (i, 0)),
        ],
        out_shape=[
            jax.ShapeDtypeStruct((_N, _F), jnp.float32),
            jax.ShapeDtypeStruct((_N, _F), jnp.float32),
        ],
        scratch_shapes=[pltpu.VMEM((1, _E), jnp.float32)],
        compiler_params=pltpu.CompilerParams(
            dimension_semantics=("arbitrary",)
        ),
    )(inputs, W)
    return disp.reshape(_N, _E, _C), comb.reshape(_N, _E, _C)
